# baseline (speedup 1.0000x reference)
import jax
import jax.numpy as jnp
from jax import lax
from jax.experimental import pallas as pl
from jax.experimental.pallas import tpu as pltpu
from jax.experimental.pallas import tpu_sc as plsc

N = 10000
D = 128
E = 320000
L = 200000
NC = 2
NS = 16
NW = NC * NS
RPA = 624
RPB = N - 15 * RPA
NPAD = N + 8

CH = 128
CPW = 80
EPAD = NW * CPW * CH
NB = 4

LNC = 56
LPW = LNC * CH
LPAD = LPW * NW

_F32 = jnp.float32
_I32 = jnp.int32


def _mesh():
    return plsc.VectorSubcoreMesh(
        core_axis_name="c", subcore_axis_name="s",
        num_cores=NC, num_subcores=NS)


def _wid():
    return lax.axis_index("s") * NC + lax.axis_index("c")


def _rows_copy(sid, copy_fn):
    @pl.when(sid != NS - 1)
    def _():
        copy_fn(sid * RPA, RPA)

    @pl.when(sid == NS - 1)
    def _():
        copy_fn((NS - 1) * RPA, RPB)


def _init_acc(cid, sid, self_hbm, zeros_hbm, acc):
    def _init(r0, nr):
        r0 = pl.multiple_of(r0, 8)

        @pl.when(cid == 0)
        def _():
            pltpu.sync_copy(self_hbm.at[pl.ds(r0, nr)], acc.at[pl.ds(r0, nr)])

        @pl.when(cid != 0)
        def _():
            pltpu.sync_copy(zeros_hbm.at[pl.ds(r0, nr)], acc.at[pl.ds(r0, nr)])

    _rows_copy(sid, _init)

    @pl.when(sid == 0)
    def _():
        pltpu.sync_copy(zeros_hbm.at[pl.ds(0, 8)], acc.at[pl.ds(N, 8)])


def _wb_acc(cid, sid, acc, out_hbm):
    def _wb(r0, nr):
        r0 = pl.multiple_of(r0, 8)
        o0 = pl.multiple_of(cid * N + r0, 8)
        pltpu.sync_copy(acc.at[pl.ds(r0, nr)], out_hbm.at[pl.ds(o0, nr)])

    _rows_copy(sid, _wb)


def _deg_body(dst2_hbm, zeros_hbm, ones_hbm, out_hbm,
              didx, ones_v, acc, s0, s1, s2, s3):
    sems = [s0, s1, s2, s3]
    cid = lax.axis_index("c")
    sid = lax.axis_index("s")
    wid = _wid()

    pltpu.sync_copy(dst2_hbm.at[pl.ds(wid * CPW, CPW)], didx)
    pltpu.sync_copy(ones_hbm, ones_v)
    _init_acc(cid, sid, zeros_hbm, zeros_hbm, acc)
    plsc.subcore_barrier()

    for b in range(NB):
        pltpu.async_copy(ones_v, acc.at[didx.at[b]], sems[b], add=True)

    @pl.loop(0, CPW // NB)
    def _outer(o):
        for b in range(NB):
            i = o * NB + b
            pltpu.make_async_copy(ones_v, acc.at[didx.at[i]], sems[b]).wait()

            @pl.when(o < CPW // NB - 1)
            def _():
                pltpu.async_copy(ones_v, acc.at[didx.at[i + NB]],
                                 sems[b], add=True)

    plsc.subcore_barrier()
    _wb_acc(cid, sid, acc, out_hbm)


def _msg_body(hs_hbm, src2_hbm, dst2_hbm, zeros_hbm, out_hbm,
              sidx, didx, r0b, r1b, acc, g0, g1, d0, d1):
    rows = [r0b, r1b]
    gsem = [g0, g1]
    dsem = [d0, d1]
    cid = lax.axis_index("c")
    sid = lax.axis_index("s")
    wid = _wid()
    cb = wid * CPW
    nblk = CPW // 8

    pltpu.sync_copy(src2_hbm.at[pl.ds(cb, CPW)], sidx)
    pltpu.sync_copy(dst2_hbm.at[pl.ds(cb, 8)], didx.at[0])
    pltpu.sync_copy(dst2_hbm.at[pl.ds(cb + 8, 8)], didx.at[1])
    for b in range(2):
        pltpu.async_copy(hs_hbm.at[sidx.at[b]], rows[b], gsem[b])

    _init_acc(cid, sid, hs_hbm, zeros_hbm, acc)
    plsc.subcore_barrier()

    @pl.loop(0, nblk // 2)
    def _outer(oo):
        for bb in range(2):
            g = oo * 2 + bb

            @pl.when(g >= 2)
            def _():
                pltpu.make_async_copy(dst2_hbm.at[pl.ds(cb + g * 8, 8)],
                                      didx.at[bb], dsem[bb]).wait()

            for s in range(8):
                i = g * 8 + s
                rb = s % 2
                pltpu.make_async_copy(hs_hbm.at[sidx.at[i]], rows[rb],
                                      gsem[rb]).wait()
                pltpu.async_copy(rows[rb], acc.at[didx.at[bb, s]],
                                 gsem[rb], add=True).wait()

                @pl.when(i + 2 < CPW)
                def _():
                    pltpu.async_copy(hs_hbm.at[sidx.at[i + 2]], rows[rb],
                                     gsem[rb])

            @pl.when(g + 2 < nblk)
            def _():
                pltpu.async_copy(dst2_hbm.at[pl.ds(cb + (g + 2) * 8, 8)],
                                 didx.at[bb], dsem[bb])

    plsc.subcore_barrier()
    _wb_acc(cid, sid, acc, out_hbm)


def _dec_body(z_hbm, srcl2_hbm, dstl2_hbm, out_hbm,
              aidx, bidx, ar0, ar1, br0, br1, p16_v, s0, s1):
    arows = [ar0, ar1]
    brows = [br0, br1]
    sems = [s0, s1]
    wid = _wid()
    cb = wid * LNC

    pltpu.sync_copy(srcl2_hbm.at[pl.ds(cb, LNC)], aidx)
    pltpu.sync_copy(dstl2_hbm.at[pl.ds(cb, LNC)], bidx)
    for b in range(2):
        pltpu.async_copy(z_hbm.at[aidx.at[b]], arows[b], sems[b])
        pltpu.async_copy(z_hbm.at[bidx.at[b]], brows[b], sems[b])

    @pl.loop(0, LNC // 2)
    def _outer(o):
        for b in range(2):
            i = o * 2 + b
            pltpu.make_async_copy(z_hbm.at[aidx.at[i]], arows[b],
                                  sems[b]).wait()
            pltpu.make_async_copy(z_hbm.at[bidx.at[i]], brows[b],
                                  sems[b]).wait()

            @pl.loop(0, CH)
            def _edge(e):
                p = arows[b][e, pl.ds(0, 16)] * brows[b][e, pl.ds(0, 16)]
                for j in range(1, 8):
                    p = p + (arows[b][e, pl.ds(16 * j, 16)]
                             * brows[b][e, pl.ds(16 * j, 16)])
                p16_v[e, :] = p

            pltpu.sync_copy(
                p16_v, out_hbm.at[pl.ds(pl.multiple_of((cb + i) * CH, 8), CH)])

            @pl.when(o < LNC // 2 - 1)
            def _():
                pltpu.async_copy(z_hbm.at[aidx.at[i + 2]], arows[b], sems[b])
                pltpu.async_copy(z_hbm.at[bidx.at[i + 2]], brows[b], sems[b])


def _ka_body(x_ref, w_ref, d0_ref, d1_ref, o_ref):
    deg = d0_ref[:, 0:1] + d1_ref[:, 0:1] + 1.0
    dinv = lax.rsqrt(deg)
    h = jnp.dot(x_ref[:], w_ref[:], preferred_element_type=_F32)
    o_ref[:] = h * dinv


def _kb_body(a0_ref, a1_ref, d0_ref, d1_ref, b_ref, w_ref, o_ref):
    deg = d0_ref[:, 0:1] + d1_ref[:, 0:1] + 1.0
    dinv = lax.rsqrt(deg)
    h = jnp.maximum((a0_ref[:] + a1_ref[:]) * dinv + b_ref[:], 0.0)
    o_ref[:] = jnp.dot(h, w_ref[:], preferred_element_type=_F32) * dinv


def _kc_body(a0_ref, a1_ref, d0_ref, d1_ref, b_ref, o_ref):
    deg = d0_ref[:, 0:1] + d1_ref[:, 0:1] + 1.0
    dinv = lax.rsqrt(deg)
    o_ref[:] = (a0_ref[:] + a1_ref[:]) * dinv + b_ref[:]


def _kd_body(p_ref, o_ref):
    o_ref[:] = jnp.sum(p_ref[:], axis=1, keepdims=True)


_RB = 2000


def _tc_ka(x, w1, d0, d1):
    return pl.pallas_call(
        _ka_body,
        grid=(N // _RB,),
        in_specs=[
            pl.BlockSpec((_RB, D), lambda i: (i, 0)),
            pl.BlockSpec((D, D), lambda i: (0, 0)),
            pl.BlockSpec((_RB, D), lambda i: (i, 0)),
            pl.BlockSpec((_RB, D), lambda i: (i, 0)),
        ],
        out_specs=pl.BlockSpec((_RB, D), lambda i: (i, 0)),
        out_shape=jax.ShapeDtypeStruct((N, D), _F32),
    )(x, w1, d0, d1)


def _tc_kb(a0, a1, d0, d1, b2d, w2):
    return pl.pallas_call(
        _kb_body,
        grid=(N // _RB,),
        in_specs=[
            pl.BlockSpec((_RB, D), lambda i: (i, 0)),
            pl.BlockSpec((_RB, D), lambda i: (i, 0)),
            pl.BlockSpec((_RB, D), lambda i: (i, 0)),
            pl.BlockSpec((_RB, D), lambda i: (i, 0)),
            pl.BlockSpec((1, D), lambda i: (0, 0)),
            pl.BlockSpec((D, D), lambda i: (0, 0)),
        ],
        out_specs=pl.BlockSpec((_RB, D), lambda i: (i, 0)),
        out_shape=jax.ShapeDtypeStruct((N, D), _F32),
    )(a0, a1, d0, d1, b2d, w2)


def _tc_kc(a0, a1, d0, d1, b2d):
    return pl.pallas_call(
        _kc_body,
        grid=(N // _RB,),
        in_specs=[
            pl.BlockSpec((_RB, D), lambda i: (i, 0)),
            pl.BlockSpec((_RB, D), lambda i: (i, 0)),
            pl.BlockSpec((_RB, D), lambda i: (i, 0)),
            pl.BlockSpec((_RB, D), lambda i: (i, 0)),
            pl.BlockSpec((1, D), lambda i: (0, 0)),
        ],
        out_specs=pl.BlockSpec((_RB, D), lambda i: (i, 0)),
        out_shape=jax.ShapeDtypeStruct((N, D), _F32),
    )(a0, a1, d0, d1, b2d)


_LB = 7168


def _tc_kd(p16):
    return pl.pallas_call(
        _kd_body,
        grid=(LPAD // _LB,),
        in_specs=[pl.BlockSpec((_LB, 16), lambda i: (i, 0))],
        out_specs=pl.BlockSpec((_LB, 1), lambda i: (i, 0)),
        out_shape=jax.ShapeDtypeStruct((LPAD, 1), _F32),
    )(p16)


def kernel(x, edge_index, edge_label_index, W1, b1, W2, b2):
    ei = edge_index.astype(_I32)
    eli = edge_label_index.astype(_I32)
    epad = EPAD - E
    src2 = jnp.concatenate([ei[0], jnp.zeros((epad,), _I32)]).reshape(-1, CH)
    dst2 = jnp.concatenate(
        [ei[1], N + (jnp.arange(epad, dtype=_I32) % 8)]).reshape(-1, CH)
    lpad = LPAD - L
    srcl2 = jnp.concatenate([eli[0], jnp.zeros((lpad,), _I32)]).reshape(-1, CH)
    dstl2 = jnp.concatenate([eli[1], jnp.zeros((lpad,), _I32)]).reshape(-1, CH)

    zeros_nd = jnp.zeros((N, D), _F32)
    ones_ch = jnp.ones((CH, D), _F32)
    b1_2d = b1.reshape(1, D)
    b2_2d = b2.reshape(1, D)

    mesh = _mesh()
    dma = pltpu.SemaphoreType.DMA

    deg_call = pl.kernel(
        _deg_body,
        out_type=jax.ShapeDtypeStruct((2 * N, D), _F32),
        mesh=mesh,
        scratch_types=[
            pltpu.VMEM((CPW, CH), _I32),
            pltpu.VMEM((CH, D), _F32),
            pltpu.MemorySpace.VMEM_SHARED((NPAD, D), _F32),
            dma, dma, dma, dma,
        ],
    )
    degp = deg_call(dst2, zeros_nd, ones_ch)
    d0, d1 = degp[:N], degp[N:]

    msg_call = pl.kernel(
        _msg_body,
        out_type=jax.ShapeDtypeStruct((2 * N, D), _F32),
        mesh=mesh,
        scratch_types=[
            pltpu.VMEM((CPW, CH), _I32),
            pltpu.VMEM((2, 8, CH), _I32),
            pltpu.VMEM((CH, D), _F32),
            pltpu.VMEM((CH, D), _F32),
            pltpu.MemorySpace.VMEM_SHARED((NPAD, D), _F32),
            dma, dma, dma, dma,
        ],
    )

    hs1 = _tc_ka(x, W1, d0, d1)
    acc1 = msg_call(hs1, src2, dst2, zeros_nd)
    hs2 = _tc_kb(acc1[:N], acc1[N:], d0, d1, b1_2d, W2)
    acc2 = msg_call(hs2, src2, dst2, zeros_nd)
    z = _tc_kc(acc2[:N], acc2[N:], d0, d1, b2_2d)

    dec_call = pl.kernel(
        _dec_body,
        out_type=jax.ShapeDtypeStruct((LPAD, 16), _F32),
        mesh=mesh,
        scratch_types=[
            pltpu.VMEM((LNC, CH), _I32),
            pltpu.VMEM((LNC, CH), _I32),
            pltpu.VMEM((CH, D), _F32),
            pltpu.VMEM((CH, D), _F32),
            pltpu.VMEM((CH, D), _F32),
            pltpu.VMEM((CH, D), _F32),
            pltpu.VMEM((CH, 16), _F32),
            dma, dma,
        ],
    )
    p16 = dec_call(z, srcl2, dstl2)
    score = _tc_kd(p16)
    return score[:L, 0]


# speedup vs baseline: 5.7119x; 5.7119x over previous
import jax
import jax.numpy as jnp
from jax import lax
from jax.experimental import pallas as pl
from jax.experimental.pallas import tpu as pltpu
from jax.experimental.pallas import tpu_sc as plsc

N = 10000
D = 128
E = 320000
L = 200000
NC = 2
NS = 16
NW = NC * NS
RPA = 624
RPB = N - 15 * RPA
NPAD = N + 512

CH = 128
CPW = 80
EPAD = NW * CPW * CH
NB = 4

LNC = 56
LPW = LNC * CH
LPAD = LPW * NW

_F32 = jnp.float32
_I32 = jnp.int32


def _mesh():
    return plsc.VectorSubcoreMesh(
        core_axis_name="c", subcore_axis_name="s",
        num_cores=NC, num_subcores=NS)


def _wid():
    return lax.axis_index("s") * NC + lax.axis_index("c")


def _rows_copy(sid, copy_fn):
    @pl.when(sid != NS - 1)
    def _():
        copy_fn(sid * RPA, RPA)

    @pl.when(sid == NS - 1)
    def _():
        copy_fn((NS - 1) * RPA, RPB)


def _init_acc(cid, sid, self_hbm, zeros_hbm, acc):
    def _init(r0, nr):
        r0 = pl.multiple_of(r0, 8)

        @pl.when(cid == 0)
        def _():
            pltpu.sync_copy(self_hbm.at[pl.ds(r0, nr)], acc.at[pl.ds(r0, nr)])

        @pl.when(cid != 0)
        def _():
            pltpu.sync_copy(zeros_hbm.at[pl.ds(r0, nr)], acc.at[pl.ds(r0, nr)])

    _rows_copy(sid, _init)

    @pl.when(sid == 0)
    def _():
        pltpu.sync_copy(zeros_hbm.at[pl.ds(0, NPAD - N)], acc.at[pl.ds(N, NPAD - N)])


def _wb_acc(cid, sid, acc, out_hbm):
    def _wb(r0, nr):
        r0 = pl.multiple_of(r0, 8)
        o0 = pl.multiple_of(cid * N + r0, 8)
        pltpu.sync_copy(acc.at[pl.ds(r0, nr)], out_hbm.at[pl.ds(o0, nr)])

    _rows_copy(sid, _wb)


def _deg_body(dst2_hbm, zeros_hbm, ones_hbm, out_hbm,
              didx, ones_v, acc, s0, s1, s2, s3):
    sems = [s0, s1, s2, s3]
    cid = lax.axis_index("c")
    sid = lax.axis_index("s")
    wid = _wid()

    pltpu.sync_copy(dst2_hbm.at[pl.ds(wid * CPW, CPW)], didx)
    pltpu.sync_copy(ones_hbm, ones_v)
    _init_acc(cid, sid, zeros_hbm, zeros_hbm, acc)
    plsc.subcore_barrier()

    for b in range(NB):
        pltpu.async_copy(ones_v, acc.at[didx.at[b]], sems[b], add=True)

    @pl.loop(0, CPW // NB)
    def _outer(o):
        for b in range(NB):
            i = o * NB + b
            pltpu.make_async_copy(ones_v, acc.at[didx.at[i]], sems[b]).wait()

            @pl.when(o < CPW // NB - 1)
            def _():
                pltpu.async_copy(ones_v, acc.at[didx.at[i + NB]],
                                 sems[b], add=True)

    plsc.subcore_barrier()
    _wb_acc(cid, sid, acc, out_hbm)


def _msg_body(hs_hbm, src2_hbm, dst2_hbm, zeros_hbm, out_hbm,
              sidx, didx, r0b, r1b, acc, g0, g1, d0, d1):
    rows = [r0b, r1b]
    gsem = [g0, g1]
    dsem = [d0, d1]
    cid = lax.axis_index("c")
    sid = lax.axis_index("s")
    wid = _wid()
    cb = wid * CPW
    nblk = CPW // 8

    pltpu.sync_copy(src2_hbm.at[pl.ds(cb, CPW)], sidx)
    pltpu.sync_copy(dst2_hbm.at[pl.ds(cb, 8)], didx.at[0])
    pltpu.sync_copy(dst2_hbm.at[pl.ds(cb + 8, 8)], didx.at[1])
    for b in range(2):
        pltpu.async_copy(hs_hbm.at[sidx.at[b]], rows[b], gsem[b])

    _init_acc(cid, sid, hs_hbm, zeros_hbm, acc)
    plsc.subcore_barrier()

    @pl.loop(0, nblk // 2)
    def _outer(oo):
        for bb in range(2):
            g = oo * 2 + bb

            @pl.when(g >= 2)
            def _():
                pltpu.make_async_copy(dst2_hbm.at[pl.ds(cb + g * 8, 8)],
                                      didx.at[bb], dsem[bb]).wait()

            for s in range(8):
                i = g * 8 + s
                rb = s % 2
                pltpu.make_async_copy(hs_hbm.at[sidx.at[i]], rows[rb],
                                      gsem[rb]).wait()
                pltpu.async_copy(rows[rb], acc.at[didx.at[bb, s]],
                                 gsem[rb], add=True).wait()

                @pl.when(i + 2 < CPW)
                def _():
                    pltpu.async_copy(hs_hbm.at[sidx.at[i + 2]], rows[rb],
                                     gsem[rb])

            @pl.when(g + 2 < nblk)
            def _():
                pltpu.async_copy(dst2_hbm.at[pl.ds(cb + (g + 2) * 8, 8)],
                                 didx.at[bb], dsem[bb])

    plsc.subcore_barrier()
    _wb_acc(cid, sid, acc, out_hbm)


def _dec_body(z_hbm, srcl2_hbm, dstl2_hbm, out_hbm,
              aidx, bidx, ar0, ar1, br0, br1, p16_v, s0, s1):
    arows = [ar0, ar1]
    brows = [br0, br1]
    sems = [s0, s1]
    wid = _wid()
    cb = wid * LNC

    pltpu.sync_copy(srcl2_hbm.at[pl.ds(cb, LNC)], aidx)
    pltpu.sync_copy(dstl2_hbm.at[pl.ds(cb, LNC)], bidx)
    for b in range(2):
        pltpu.async_copy(z_hbm.at[aidx.at[b]], arows[b], sems[b])
        pltpu.async_copy(z_hbm.at[bidx.at[b]], brows[b], sems[b])

    @pl.loop(0, LNC // 2)
    def _outer(o):
        for b in range(2):
            i = o * 2 + b
            pltpu.make_async_copy(z_hbm.at[aidx.at[i]], arows[b],
                                  sems[b]).wait()
            pltpu.make_async_copy(z_hbm.at[bidx.at[i]], brows[b],
                                  sems[b]).wait()

            @pl.loop(0, CH)
            def _edge(e):
                p = arows[b][e, pl.ds(0, 16)] * brows[b][e, pl.ds(0, 16)]
                for j in range(1, 8):
                    p = p + (arows[b][e, pl.ds(16 * j, 16)]
                             * brows[b][e, pl.ds(16 * j, 16)])
                p16_v[e, :] = p

            pltpu.sync_copy(
                p16_v, out_hbm.at[pl.ds(pl.multiple_of((cb + i) * CH, 8), CH)])

            @pl.when(o < LNC // 2 - 1)
            def _():
                pltpu.async_copy(z_hbm.at[aidx.at[i + 2]], arows[b], sems[b])
                pltpu.async_copy(z_hbm.at[bidx.at[i + 2]], brows[b], sems[b])


def _ka_body(x_ref, w_ref, d0_ref, d1_ref, o_ref):
    deg = d0_ref[:, 0:1] + d1_ref[:, 0:1] + 1.0
    dinv = lax.rsqrt(deg)
    h = jnp.dot(x_ref[:], w_ref[:], preferred_element_type=_F32)
    o_ref[:] = h * dinv


def _kb_body(a0_ref, a1_ref, d0_ref, d1_ref, b_ref, w_ref, o_ref):
    deg = d0_ref[:, 0:1] + d1_ref[:, 0:1] + 1.0
    dinv = lax.rsqrt(deg)
    h = jnp.maximum((a0_ref[:] + a1_ref[:]) * dinv + b_ref[:], 0.0)
    o_ref[:] = jnp.dot(h, w_ref[:], preferred_element_type=_F32) * dinv


def _kc_body(a0_ref, a1_ref, d0_ref, d1_ref, b_ref, o_ref):
    deg = d0_ref[:, 0:1] + d1_ref[:, 0:1] + 1.0
    dinv = lax.rsqrt(deg)
    o_ref[:] = (a0_ref[:] + a1_ref[:]) * dinv + b_ref[:]


def _kd_body(p_ref, o_ref):
    o_ref[:] = jnp.sum(p_ref[:], axis=1, keepdims=True)


_RB = 2000


def _tc_ka(x, w1, d0, d1):
    return pl.pallas_call(
        _ka_body,
        grid=(N // _RB,),
        in_specs=[
            pl.BlockSpec((_RB, D), lambda i: (i, 0)),
            pl.BlockSpec((D, D), lambda i: (0, 0)),
            pl.BlockSpec((_RB, D), lambda i: (i, 0)),
            pl.BlockSpec((_RB, D), lambda i: (i, 0)),
        ],
        out_specs=pl.BlockSpec((_RB, D), lambda i: (i, 0)),
        out_shape=jax.ShapeDtypeStruct((N, D), _F32),
    )(x, w1, d0, d1)


def _tc_kb(a0, a1, d0, d1, b2d, w2):
    return pl.pallas_call(
        _kb_body,
        grid=(N // _RB,),
        in_specs=[
            pl.BlockSpec((_RB, D), lambda i: (i, 0)),
            pl.BlockSpec((_RB, D), lambda i: (i, 0)),
            pl.BlockSpec((_RB, D), lambda i: (i, 0)),
            pl.BlockSpec((_RB, D), lambda i: (i, 0)),
            pl.BlockSpec((1, D), lambda i: (0, 0)),
            pl.BlockSpec((D, D), lambda i: (0, 0)),
        ],
        out_specs=pl.BlockSpec((_RB, D), lambda i: (i, 0)),
        out_shape=jax.ShapeDtypeStruct((N, D), _F32),
    )(a0, a1, d0, d1, b2d, w2)


def _tc_kc(a0, a1, d0, d1, b2d):
    return pl.pallas_call(
        _kc_body,
        grid=(N // _RB,),
        in_specs=[
            pl.BlockSpec((_RB, D), lambda i: (i, 0)),
            pl.BlockSpec((_RB, D), lambda i: (i, 0)),
            pl.BlockSpec((_RB, D), lambda i: (i, 0)),
            pl.BlockSpec((_RB, D), lambda i: (i, 0)),
            pl.BlockSpec((1, D), lambda i: (0, 0)),
        ],
        out_specs=pl.BlockSpec((_RB, D), lambda i: (i, 0)),
        out_shape=jax.ShapeDtypeStruct((N, D), _F32),
    )(a0, a1, d0, d1, b2d)


_LB = 7168


def _tc_kd(p16):
    return pl.pallas_call(
        _kd_body,
        grid=(LPAD // _LB,),
        in_specs=[pl.BlockSpec((_LB, 16), lambda i: (i, 0))],
        out_specs=pl.BlockSpec((_LB, 1), lambda i: (i, 0)),
        out_shape=jax.ShapeDtypeStruct((LPAD, 1), _F32),
    )(p16)


def kernel(x, edge_index, edge_label_index, W1, b1, W2, b2):
    ei = edge_index.astype(_I32)
    eli = edge_label_index.astype(_I32)
    epad = EPAD - E
    spread_e = jnp.arange(epad, dtype=_I32)
    src2 = jnp.concatenate([ei[0], spread_e % N]).reshape(-1, CH)
    dst2 = jnp.concatenate(
        [ei[1], N + (spread_e % (NPAD - N))]).reshape(-1, CH)
    lpad = LPAD - L
    spread_l = jnp.arange(lpad, dtype=_I32)
    srcl2 = jnp.concatenate([eli[0], spread_l % N]).reshape(-1, CH)
    dstl2 = jnp.concatenate(
        [eli[1], (spread_l * 127) % N]).reshape(-1, CH)

    zeros_nd = jnp.zeros((N, D), _F32)
    ones_ch = jnp.ones((CH, D), _F32)
    b1_2d = b1.reshape(1, D)
    b2_2d = b2.reshape(1, D)

    mesh = _mesh()
    dma = pltpu.SemaphoreType.DMA

    deg_call = pl.kernel(
        _deg_body,
        out_type=jax.ShapeDtypeStruct((2 * N, D), _F32),
        mesh=mesh,
        scratch_types=[
            pltpu.VMEM((CPW, CH), _I32),
            pltpu.VMEM((CH, D), _F32),
            pltpu.MemorySpace.VMEM_SHARED((NPAD, D), _F32),
            dma, dma, dma, dma,
        ],
    )
    degp = deg_call(dst2, zeros_nd, ones_ch)
    d0, d1 = degp[:N], degp[N:]

    msg_call = pl.kernel(
        _msg_body,
        out_type=jax.ShapeDtypeStruct((2 * N, D), _F32),
        mesh=mesh,
        scratch_types=[
            pltpu.VMEM((CPW, CH), _I32),
            pltpu.VMEM((2, 8, CH), _I32),
            pltpu.VMEM((CH, D), _F32),
            pltpu.VMEM((CH, D), _F32),
            pltpu.MemorySpace.VMEM_SHARED((NPAD, D), _F32),
            dma, dma, dma, dma,
        ],
    )

    hs1 = _tc_ka(x, W1, d0, d1)
    acc1 = msg_call(hs1, src2, dst2, zeros_nd)
    hs2 = _tc_kb(acc1[:N], acc1[N:], d0, d1, b1_2d, W2)
    acc2 = msg_call(hs2, src2, dst2, zeros_nd)
    z = _tc_kc(acc2[:N], acc2[N:], d0, d1, b2_2d)

    dec_call = pl.kernel(
        _dec_body,
        out_type=jax.ShapeDtypeStruct((LPAD, 16), _F32),
        mesh=mesh,
        scratch_types=[
            pltpu.VMEM((LNC, CH), _I32),
            pltpu.VMEM((LNC, CH), _I32),
            pltpu.VMEM((CH, D), _F32),
            pltpu.VMEM((CH, D), _F32),
            pltpu.VMEM((CH, D), _F32),
            pltpu.VMEM((CH, D), _F32),
            pltpu.VMEM((CH, 16), _F32),
            dma, dma,
        ],
    )
    p16 = dec_call(z, srcl2, dstl2)
    score = _tc_kd(p16)
    return score[:L, 0]


# speedup vs baseline: 5.9086x; 1.0344x over previous
import jax
import jax.numpy as jnp
from jax import lax
from jax.experimental import pallas as pl
from jax.experimental.pallas import tpu as pltpu
from jax.experimental.pallas import tpu_sc as plsc

N = 10000
D = 128
E = 320000
L = 200000
NC = 2
NS = 16
NW = NC * NS
RPA = 624
RPB = N - 15 * RPA
NPAD = N + 512

CH = 128
CPW = 80
EPAD = NW * CPW * CH
NB = 4

LNC = 56
LPW = LNC * CH
LPAD = LPW * NW

_F32 = jnp.float32
_I32 = jnp.int32


def _mesh():
    return plsc.VectorSubcoreMesh(
        core_axis_name="c", subcore_axis_name="s",
        num_cores=NC, num_subcores=NS)


def _wid():
    return lax.axis_index("s") * NC + lax.axis_index("c")


def _rows_copy(sid, copy_fn):
    @pl.when(sid != NS - 1)
    def _():
        copy_fn(sid * RPA, RPA)

    @pl.when(sid == NS - 1)
    def _():
        copy_fn((NS - 1) * RPA, RPB)


def _init_acc(cid, sid, self_hbm, zeros_hbm, acc):
    def _init(r0, nr):
        r0 = pl.multiple_of(r0, 8)

        @pl.when(cid == 0)
        def _():
            pltpu.sync_copy(self_hbm.at[pl.ds(r0, nr)], acc.at[pl.ds(r0, nr)])

        @pl.when(cid != 0)
        def _():
            pltpu.sync_copy(zeros_hbm.at[pl.ds(r0, nr)], acc.at[pl.ds(r0, nr)])

    _rows_copy(sid, _init)

    @pl.when(sid == 0)
    def _():
        pltpu.sync_copy(zeros_hbm.at[pl.ds(0, NPAD - N)], acc.at[pl.ds(N, NPAD - N)])


def _wb_acc(cid, sid, acc, out_hbm):
    def _wb(r0, nr):
        r0 = pl.multiple_of(r0, 8)
        o0 = pl.multiple_of(cid * N + r0, 8)
        pltpu.sync_copy(acc.at[pl.ds(r0, nr)], out_hbm.at[pl.ds(o0, nr)])

    _rows_copy(sid, _wb)


def _deg_body(dst2_hbm, zeros_hbm, ones_hbm, out_hbm,
              didx, ones_v, acc, s0, s1, s2, s3):
    sems = [s0, s1, s2, s3]
    cid = lax.axis_index("c")
    sid = lax.axis_index("s")
    wid = _wid()

    pltpu.sync_copy(dst2_hbm.at[pl.ds(wid * CPW, CPW)], didx)
    pltpu.sync_copy(ones_hbm, ones_v)
    _init_acc(cid, sid, zeros_hbm, zeros_hbm, acc)
    plsc.subcore_barrier()

    for b in range(NB):
        pltpu.async_copy(ones_v, acc.at[didx.at[b]], sems[b], add=True)

    @pl.loop(0, CPW // NB)
    def _outer(o):
        for b in range(NB):
            i = o * NB + b
            pltpu.make_async_copy(ones_v, acc.at[didx.at[i]], sems[b]).wait()

            @pl.when(o < CPW // NB - 1)
            def _():
                pltpu.async_copy(ones_v, acc.at[didx.at[i + NB]],
                                 sems[b], add=True)

    plsc.subcore_barrier()
    _wb_acc(cid, sid, acc, out_hbm)


def _msg_body(hs_hbm, src2_hbm, dst2_hbm, zeros_hbm, out_hbm,
              sidx, didx, r0b, r1b, acc, g0, g1, d0, d1):
    rows = [r0b, r1b]
    gsem = [g0, g1]
    dsem = [d0, d1]
    cid = lax.axis_index("c")
    sid = lax.axis_index("s")
    wid = _wid()
    cb = wid * CPW
    nblk = CPW // 8

    pltpu.sync_copy(src2_hbm.at[pl.ds(cb, CPW)], sidx)
    pltpu.sync_copy(dst2_hbm.at[pl.ds(cb, 8)], didx.at[0])
    pltpu.sync_copy(dst2_hbm.at[pl.ds(cb + 8, 8)], didx.at[1])
    for b in range(2):
        pltpu.async_copy(hs_hbm.at[sidx.at[b]], rows[b], gsem[b])

    _init_acc(cid, sid, hs_hbm, zeros_hbm, acc)
    plsc.subcore_barrier()

    @pl.loop(0, nblk // 2)
    def _outer(oo):
        for bb in range(2):
            g = oo * 2 + bb

            @pl.when(g >= 2)
            def _():
                pltpu.make_async_copy(dst2_hbm.at[pl.ds(cb + g * 8, 8)],
                                      didx.at[bb], dsem[bb]).wait()

            for s in range(8):
                i = g * 8 + s
                rb = s % 2
                pltpu.make_async_copy(hs_hbm.at[sidx.at[i]], rows[rb],
                                      gsem[rb]).wait()
                pltpu.async_copy(rows[rb], acc.at[didx.at[bb, s]],
                                 gsem[rb], add=True).wait()

                @pl.when(i + 2 < CPW)
                def _():
                    pltpu.async_copy(hs_hbm.at[sidx.at[i + 2]], rows[rb],
                                     gsem[rb])

            @pl.when(g + 2 < nblk)
            def _():
                pltpu.async_copy(dst2_hbm.at[pl.ds(cb + (g + 2) * 8, 8)],
                                 didx.at[bb], dsem[bb])

    plsc.subcore_barrier()
    _wb_acc(cid, sid, acc, out_hbm)


def _dec_body(z_hbm, srcl2_hbm, dstl2_hbm, out_hbm,
              aidx, bidx, ar0, ar1, br0, br1, p16_v, s0, s1):
    arows = [ar0, ar1]
    brows = [br0, br1]
    sems = [s0, s1]
    wid = _wid()
    cb = wid * LNC

    pltpu.sync_copy(srcl2_hbm.at[pl.ds(cb, LNC)], aidx)
    pltpu.sync_copy(dstl2_hbm.at[pl.ds(cb, LNC)], bidx)
    for b in range(2):
        pltpu.async_copy(z_hbm.at[aidx.at[b]], arows[b], sems[b])
        pltpu.async_copy(z_hbm.at[bidx.at[b]], brows[b], sems[b])

    @pl.loop(0, LNC // 2)
    def _outer(o):
        for b in range(2):
            i = o * 2 + b
            pltpu.make_async_copy(z_hbm.at[aidx.at[i]], arows[b],
                                  sems[b]).wait()
            pltpu.make_async_copy(z_hbm.at[bidx.at[i]], brows[b],
                                  sems[b]).wait()

            @pl.loop(0, CH)
            def _edge(e):
                p = arows[b][e, pl.ds(0, 16)] * brows[b][e, pl.ds(0, 16)]
                for j in range(1, 8):
                    p = p + (arows[b][e, pl.ds(16 * j, 16)]
                             * brows[b][e, pl.ds(16 * j, 16)])
                p16_v[e, :] = p

            pltpu.sync_copy(
                p16_v, out_hbm.at[pl.ds(pl.multiple_of((cb + i) * CH, 8), CH)])

            @pl.when(o < LNC // 2 - 1)
            def _():
                pltpu.async_copy(z_hbm.at[aidx.at[i + 2]], arows[b], sems[b])
                pltpu.async_copy(z_hbm.at[bidx.at[i + 2]], brows[b], sems[b])


def _ka_body(x_ref, w_ref, d0_ref, d1_ref, o_ref):
    deg = d0_ref[:, 0:1] + d1_ref[:, 0:1] + 1.0
    dinv = lax.rsqrt(deg)
    h = jnp.dot(x_ref[:], w_ref[:], preferred_element_type=_F32)
    o_ref[:] = h * dinv


def _kb_body(a0_ref, a1_ref, d0_ref, d1_ref, b_ref, w_ref, o_ref):
    deg = d0_ref[:, 0:1] + d1_ref[:, 0:1] + 1.0
    dinv = lax.rsqrt(deg)
    h = jnp.maximum((a0_ref[:] + a1_ref[:]) * dinv + b_ref[:], 0.0)
    o_ref[:] = jnp.dot(h, w_ref[:], preferred_element_type=_F32) * dinv


def _kc_body(a0_ref, a1_ref, d0_ref, d1_ref, b_ref, o_ref):
    deg = d0_ref[:, 0:1] + d1_ref[:, 0:1] + 1.0
    dinv = lax.rsqrt(deg)
    o_ref[:] = (a0_ref[:] + a1_ref[:]) * dinv + b_ref[:]


def _kd_body(p_ref, o_ref):
    o_ref[:] = jnp.sum(p_ref[:], axis=1, keepdims=True)


_RB = 2000


_NBLK = N // _RB


def _tc_ka(x, w1, degp):
    return pl.pallas_call(
        _ka_body,
        grid=(_NBLK,),
        in_specs=[
            pl.BlockSpec((_RB, D), lambda i: (i, 0)),
            pl.BlockSpec((D, D), lambda i: (0, 0)),
            pl.BlockSpec((_RB, D), lambda i: (i, 0)),
            pl.BlockSpec((_RB, D), lambda i: (i + _NBLK, 0)),
        ],
        out_specs=pl.BlockSpec((_RB, D), lambda i: (i, 0)),
        out_shape=jax.ShapeDtypeStruct((N, D), _F32),
    )(x, w1, degp, degp)


def _tc_kb(accp, degp, b2d, w2):
    return pl.pallas_call(
        _kb_body,
        grid=(_NBLK,),
        in_specs=[
            pl.BlockSpec((_RB, D), lambda i: (i, 0)),
            pl.BlockSpec((_RB, D), lambda i: (i + _NBLK, 0)),
            pl.BlockSpec((_RB, D), lambda i: (i, 0)),
            pl.BlockSpec((_RB, D), lambda i: (i + _NBLK, 0)),
            pl.BlockSpec((1, D), lambda i: (0, 0)),
            pl.BlockSpec((D, D), lambda i: (0, 0)),
        ],
        out_specs=pl.BlockSpec((_RB, D), lambda i: (i, 0)),
        out_shape=jax.ShapeDtypeStruct((N, D), _F32),
    )(accp, accp, degp, degp, b2d, w2)


def _tc_kc(accp, degp, b2d):
    return pl.pallas_call(
        _kc_body,
        grid=(_NBLK,),
        in_specs=[
            pl.BlockSpec((_RB, D), lambda i: (i, 0)),
            pl.BlockSpec((_RB, D), lambda i: (i + _NBLK, 0)),
            pl.BlockSpec((_RB, D), lambda i: (i, 0)),
            pl.BlockSpec((_RB, D), lambda i: (i + _NBLK, 0)),
            pl.BlockSpec((1, D), lambda i: (0, 0)),
        ],
        out_specs=pl.BlockSpec((_RB, D), lambda i: (i, 0)),
        out_shape=jax.ShapeDtypeStruct((N, D), _F32),
    )(accp, accp, degp, degp, b2d)


_LB = 7168


def _tc_kd(p16):
    return pl.pallas_call(
        _kd_body,
        grid=(LPAD // _LB,),
        in_specs=[pl.BlockSpec((_LB, 16), lambda i: (i, 0))],
        out_specs=pl.BlockSpec((_LB, 1), lambda i: (i, 0)),
        out_shape=jax.ShapeDtypeStruct((LPAD, 1), _F32),
    )(p16)


def kernel(x, edge_index, edge_label_index, W1, b1, W2, b2):
    ei = edge_index.astype(_I32)
    eli = edge_label_index.astype(_I32)
    epad = EPAD - E
    spread_e = jnp.arange(epad, dtype=_I32)
    src2 = jnp.concatenate([ei[0], spread_e % N]).reshape(-1, CH)
    dst2 = jnp.concatenate(
        [ei[1], N + (spread_e % (NPAD - N))]).reshape(-1, CH)
    lpad = LPAD - L
    spread_l = jnp.arange(lpad, dtype=_I32)
    srcl2 = jnp.concatenate([eli[0], spread_l % N]).reshape(-1, CH)
    dstl2 = jnp.concatenate(
        [eli[1], (spread_l * 127) % N]).reshape(-1, CH)

    zeros_nd = jnp.zeros((N, D), _F32)
    ones_ch = jnp.ones((CH, D), _F32)
    b1_2d = b1.reshape(1, D)
    b2_2d = b2.reshape(1, D)

    mesh = _mesh()
    dma = pltpu.SemaphoreType.DMA

    deg_call = pl.kernel(
        _deg_body,
        out_type=jax.ShapeDtypeStruct((2 * N, D), _F32),
        mesh=mesh,
        scratch_types=[
            pltpu.VMEM((CPW, CH), _I32),
            pltpu.VMEM((CH, D), _F32),
            pltpu.MemorySpace.VMEM_SHARED((NPAD, D), _F32),
            dma, dma, dma, dma,
        ],
    )
    degp = deg_call(dst2, zeros_nd, ones_ch)

    msg_call = pl.kernel(
        _msg_body,
        out_type=jax.ShapeDtypeStruct((2 * N, D), _F32),
        mesh=mesh,
        scratch_types=[
            pltpu.VMEM((CPW, CH), _I32),
            pltpu.VMEM((2, 8, CH), _I32),
            pltpu.VMEM((CH, D), _F32),
            pltpu.VMEM((CH, D), _F32),
            pltpu.MemorySpace.VMEM_SHARED((NPAD, D), _F32),
            dma, dma, dma, dma,
        ],
    )

    hs1 = _tc_ka(x, W1, degp)
    acc1 = msg_call(hs1, src2, dst2, zeros_nd)
    hs2 = _tc_kb(acc1, degp, b1_2d, W2)
    acc2 = msg_call(hs2, src2, dst2, zeros_nd)
    z = _tc_kc(acc2, degp, b2_2d)

    dec_call = pl.kernel(
        _dec_body,
        out_type=jax.ShapeDtypeStruct((LPAD, 16), _F32),
        mesh=mesh,
        scratch_types=[
            pltpu.VMEM((LNC, CH), _I32),
            pltpu.VMEM((LNC, CH), _I32),
            pltpu.VMEM((CH, D), _F32),
            pltpu.VMEM((CH, D), _F32),
            pltpu.VMEM((CH, D), _F32),
            pltpu.VMEM((CH, D), _F32),
            pltpu.VMEM((CH, 16), _F32),
            dma, dma,
        ],
    )
    p16 = dec_call(z, srcl2, dstl2)
    score = _tc_kd(p16)
    return score[:L, 0]
